# baseline (speedup 1.0000x reference)
import functools

import jax
import jax.numpy as jnp
from jax import lax
from jax.experimental import pallas as pl
from jax.experimental.pallas import tpu as pltpu
from jax.experimental.pallas import tpu_sc as plsc

_SUB = 128


@functools.partial(jax.jit, static_argnames=("n_rows", "embed_dim"))
def _sc_gather(idx_grouped, table, n_rows, embed_dim):
    info = plsc.get_sparse_core_info()
    nc, ns = info.num_cores, info.num_subcores
    nw = nc * ns
    b_per_w = n_rows // nw
    n_sub = b_per_w // _SUB

    mesh = plsc.VectorSubcoreMesh(core_axis_name="c", subcore_axis_name="s")

    @functools.partial(
        pl.kernel,
        out_type=jax.ShapeDtypeStruct((n_rows, embed_dim), jnp.float32),
        mesh=mesh,
        scratch_types=[
            pltpu.VMEM((n_sub, _SUB), jnp.int32),
            pltpu.VMEM((2, _SUB, embed_dim), jnp.float32),
            pltpu.SemaphoreType.DMA,
            pltpu.SemaphoreType.DMA,
        ],
        compiler_params=pltpu.CompilerParams(use_tc_tiling_on_sc=False),
    )
    def body(idx_hbm, table_hbm, out_hbm, idx_v, rows_v, gsem0, gsem1):
        wid = lax.axis_index("s") * nc + lax.axis_index("c")
        base = wid * b_per_w
        pltpu.sync_copy(idx_hbm.at[wid], idx_v)

        gsems = (gsem0, gsem1)
        pltpu.async_copy(table_hbm.at[idx_v.at[0]], rows_v.at[0], gsems[0])

        def step(t, carry):
            for b in range(2):
                j = t * 2 + b
                nb = 1 - b

                @pl.when(j + 1 < n_sub)
                def _start_next():
                    pltpu.async_copy(
                        table_hbm.at[idx_v.at[j + 1]], rows_v.at[nb], gsems[nb]
                    )

                pltpu.make_async_copy(
                    table_hbm.at[idx_v.at[j]], rows_v.at[b], gsems[b]
                ).wait()
                pltpu.sync_copy(
                    rows_v.at[b], out_hbm.at[pl.ds(base + j * _SUB, _SUB)]
                )
            return carry

        lax.fori_loop(0, n_sub // 2, step, 0)

    return body(idx_grouped, table)


def kernel(input_ids, embedding_weight):
    batch, hist = input_ids.shape
    vocab, embed_dim = embedding_weight.shape
    n_rows = batch * hist

    info = plsc.get_sparse_core_info()
    nw = info.num_cores * info.num_subcores
    b_per_w = n_rows // nw

    idx_grouped = input_ids.astype(jnp.int32).reshape(nw, b_per_w // _SUB, _SUB)
    out = _sc_gather(idx_grouped, embedding_weight, n_rows, embed_dim)
    return out.reshape(batch, hist, embed_dim)


# speedup vs baseline: 1.0097x; 1.0097x over previous
import functools

import jax
import jax.numpy as jnp
from jax import lax
from jax.experimental import pallas as pl
from jax.experimental.pallas import tpu as pltpu
from jax.experimental.pallas import tpu_sc as plsc

_SUB = 128


@functools.partial(jax.jit, static_argnames=("n_rows", "embed_dim"))
def _sc_gather(idx_grouped, table, n_rows, embed_dim):
    info = plsc.get_sparse_core_info()
    nc, ns = info.num_cores, info.num_subcores
    nw = nc * ns
    b_per_w = n_rows // nw
    n_sub = b_per_w // _SUB

    mesh = plsc.VectorSubcoreMesh(core_axis_name="c", subcore_axis_name="s")

    nbuf = 5
    assert n_sub % nbuf == 0

    @functools.partial(
        pl.kernel,
        out_type=jax.ShapeDtypeStruct((n_rows, embed_dim), jnp.float32),
        mesh=mesh,
        scratch_types=[
            pltpu.VMEM((n_sub, _SUB), jnp.int32),
            pltpu.VMEM((nbuf, _SUB, embed_dim), jnp.float32),
            [pltpu.SemaphoreType.DMA] * nbuf,
            [pltpu.SemaphoreType.DMA] * nbuf,
        ],
        compiler_params=pltpu.CompilerParams(use_tc_tiling_on_sc=False),
    )
    def body(idx_hbm, table_hbm, out_hbm, idx_v, rows_v, gsems, wsems):
        wid = lax.axis_index("s") * nc + lax.axis_index("c")
        base = wid * b_per_w
        pltpu.sync_copy(idx_hbm.at[wid], idx_v)

        def start_gather(j, b):
            pltpu.async_copy(table_hbm.at[idx_v.at[j]], rows_v.at[b], gsems[b])

        for b in range(nbuf):
            start_gather(b, b)

        def step(t, carry):
            for b in range(nbuf):
                j = t * nbuf + b
                pb = (b - 1) % nbuf
                g = j - 1 + nbuf

                @pl.when((j >= 1) & (g < n_sub))
                def _refill():
                    pltpu.make_async_copy(
                        rows_v.at[pb],
                        out_hbm.at[pl.ds(base + (g - nbuf) * _SUB, _SUB)],
                        wsems[pb],
                    ).wait()
                    start_gather(g, pb)

                pltpu.make_async_copy(
                    table_hbm.at[idx_v.at[j]], rows_v.at[b], gsems[b]
                ).wait()
                pltpu.async_copy(
                    rows_v.at[b],
                    out_hbm.at[pl.ds(base + j * _SUB, _SUB)],
                    wsems[b],
                )
            return carry

        lax.fori_loop(0, n_sub // nbuf, step, 0)

        for b in range(nbuf):
            j = n_sub - nbuf + b
            pltpu.make_async_copy(
                rows_v.at[b % nbuf],
                out_hbm.at[pl.ds(base + j * _SUB, _SUB)],
                wsems[j % nbuf],
            ).wait()

    return body(idx_grouped, table)


def kernel(input_ids, embedding_weight):
    batch, hist = input_ids.shape
    vocab, embed_dim = embedding_weight.shape
    n_rows = batch * hist

    info = plsc.get_sparse_core_info()
    nw = info.num_cores * info.num_subcores
    b_per_w = n_rows // nw

    idx_grouped = input_ids.astype(jnp.int32).reshape(nw, b_per_w // _SUB, _SUB)
    out = _sc_gather(idx_grouped, embedding_weight, n_rows, embed_dim)
    return out.reshape(batch, hist, embed_dim)
